# baseline (speedup 1.0000x reference)
import functools

import jax
import jax.numpy as jnp
from jax import lax
from jax.experimental import pallas as pl
from jax.experimental.pallas import tpu as pltpu
from jax.experimental.pallas import tpu_sc as plsc

INPUT_SIZE = 1000000
HIDDEN = 64
BATCH = 4096
HIST = 200

_NC = 2
_NS = 16
_NW = _NC * _NS
_BPW = BATCH // _NW
_CHUNK = 100
_CPR = HIST // _CHUNK
_CPW = _BPW * _CPR


def _pool_body(x_hbm, emb_hbm, out_hbm, idx_v, rows_v, out_v, sem):
    wid = lax.axis_index("c") * _NS + lax.axis_index("s")
    pltpu.sync_copy(x_hbm.at[pl.ds(wid * _CPW, _CPW)], idx_v)

    inv = jnp.float32(1.0 / HIST)

    def row_body(b, carry):
        cp0 = pltpu.async_copy(
            emb_hbm.at[idx_v.at[2 * b]], rows_v.at[pl.ds(0, _CHUNK)], sem)
        cp1 = pltpu.async_copy(
            emb_hbm.at[idx_v.at[2 * b + 1]], rows_v.at[pl.ds(_CHUNK, _CHUNK)], sem)
        cp0.wait()
        cp1.wait()

        zero = jnp.zeros((16,), jnp.float32)

        def acc_body(j, accs):
            a = list(accs)
            for jj in range(4):
                r = 4 * j + jj
                for c in range(4):
                    a[c] = a[c] + rows_v[r, pl.ds(c * 16, 16)]
            return tuple(a)

        accs = lax.fori_loop(0, HIST // 4, acc_body, (zero, zero, zero, zero))
        for c in range(4):
            out_v[b, pl.ds(c * 16, 16)] = accs[c] * inv
        return carry

    lax.fori_loop(0, _BPW, row_body, 0)
    pltpu.sync_copy(out_v, out_hbm.at[pl.ds(wid * _BPW, _BPW)])


_pool = functools.partial(
    pl.kernel,
    mesh=plsc.VectorSubcoreMesh(core_axis_name="c", subcore_axis_name="s"),
    out_type=jax.ShapeDtypeStruct((BATCH, HIDDEN), jnp.float32),
    scratch_types=[
        pltpu.VMEM((_CPW, _CHUNK), jnp.int32),
        pltpu.VMEM((HIST, HIDDEN), jnp.float32),
        pltpu.VMEM((_BPW, HIDDEN), jnp.float32),
        pltpu.SemaphoreType.DMA,
    ],
    compiler_params=pltpu.CompilerParams(use_tc_tiling_on_sc=False),
)(_pool_body)


def _mlp_body(m_ref, w1_ref, b1_ref, w2_ref, b2_ref, o_ref):
    m = m_ref[...]
    h = jax.lax.dot_general(
        m, w1_ref[...], (((1,), (1,)), ((), ())),
        preferred_element_type=jnp.float32,
        precision=jax.lax.Precision.HIGHEST) + b1_ref[...]
    h = jnp.maximum(h, 0.0)
    o = jax.lax.dot_general(
        h, w2_ref[...], (((1,), (1,)), ((), ())),
        preferred_element_type=jnp.float32,
        precision=jax.lax.Precision.HIGHEST) + b2_ref[...]
    mx = jnp.max(o, axis=1, keepdims=True)
    lse = jnp.log(jnp.sum(jnp.exp(o - mx), axis=1, keepdims=True)) + mx
    o_ref[...] = o - lse


_mlp = pl.pallas_call(
    _mlp_body,
    out_shape=jax.ShapeDtypeStruct((BATCH, 2), jnp.float32),
)


def kernel(x, emb, W1, b1, W2, b2):
    pooled = _pool(x.reshape(BATCH * HIST // _CHUNK, _CHUNK), emb)
    return _mlp(pooled, W1, b1.reshape(1, HIDDEN), W2, b2.reshape(1, 2))


# speedup vs baseline: 1.1321x; 1.1321x over previous
import functools

import jax
import jax.numpy as jnp
from jax import lax
from jax.experimental import pallas as pl
from jax.experimental.pallas import tpu as pltpu
from jax.experimental.pallas import tpu_sc as plsc

INPUT_SIZE = 1000000
HIDDEN = 64
BATCH = 4096
HIST = 200

_NC = 2
_NS = 16
_NW = _NC * _NS
_BPW = BATCH // _NW
_CHUNK = 100
_CPR = HIST // _CHUNK
_CPW = _BPW * _CPR


def _pool_body(x_hbm, emb_hbm, out_hbm, idx_v, rows_v, out_v, sem0, sem1):
    wid = lax.axis_index("c") * _NS + lax.axis_index("s")
    pltpu.sync_copy(x_hbm.at[pl.ds(wid * _CPW, _CPW)], idx_v)

    sems = (sem0, sem1)
    inv = jnp.float32(1.0 / HIST)

    def issue(row, buf):
        for c in range(_CPR):
            pltpu.async_copy(
                emb_hbm.at[idx_v.at[_CPR * row + c]],
                rows_v.at[buf].at[pl.ds(c * _CHUNK, _CHUNK)],
                sems[buf])

    def wait(buf):
        pltpu.make_async_copy(
            emb_hbm.at[pl.ds(0, HIST)], rows_v.at[buf], sems[buf]).wait()

    issue(0, 0)
    issue(1, 1)

    @pl.loop(0, _BPW, step=2)
    def _(b):
        for p in range(2):
            row = b + p
            wait(p)

            zero = jnp.zeros((16,), jnp.float32)

            def acc_body(j, accs):
                a = list(accs)
                for jj in range(4):
                    r = 4 * j + jj
                    for c in range(4):
                        a[c] = a[c] + rows_v[p, r, pl.ds(c * 16, 16)]
                return tuple(a)

            accs = lax.fori_loop(0, HIST // 4, acc_body, (zero,) * 4)
            for c in range(4):
                out_v[row, pl.ds(c * 16, 16)] = accs[c] * inv

            nxt = row + 2
            @pl.when(nxt < _BPW)
            def _():
                issue(nxt, p)

    pltpu.sync_copy(out_v, out_hbm.at[pl.ds(wid * _BPW, _BPW)])


_pool = functools.partial(
    pl.kernel,
    mesh=plsc.VectorSubcoreMesh(core_axis_name="c", subcore_axis_name="s"),
    out_type=jax.ShapeDtypeStruct((BATCH, HIDDEN), jnp.float32),
    scratch_types=[
        pltpu.VMEM((_CPW, _CHUNK), jnp.int32),
        pltpu.VMEM((2, HIST, HIDDEN), jnp.float32),
        pltpu.VMEM((_BPW, HIDDEN), jnp.float32),
        pltpu.SemaphoreType.DMA,
        pltpu.SemaphoreType.DMA,
    ],
    compiler_params=pltpu.CompilerParams(use_tc_tiling_on_sc=False),
)(_pool_body)


def _mlp_body(m_ref, w1_ref, b1_ref, w2_ref, b2_ref, o_ref):
    m = m_ref[...]
    h = jax.lax.dot_general(
        m, w1_ref[...], (((1,), (1,)), ((), ())),
        preferred_element_type=jnp.float32,
        precision=jax.lax.Precision.HIGHEST) + b1_ref[...]
    h = jnp.maximum(h, 0.0)
    o = jax.lax.dot_general(
        h, w2_ref[...], (((1,), (1,)), ((), ())),
        preferred_element_type=jnp.float32,
        precision=jax.lax.Precision.HIGHEST) + b2_ref[...]
    mx = jnp.max(o, axis=1, keepdims=True)
    lse = jnp.log(jnp.sum(jnp.exp(o - mx), axis=1, keepdims=True)) + mx
    o_ref[...] = o - lse


_mlp = pl.pallas_call(
    _mlp_body,
    out_shape=jax.ShapeDtypeStruct((BATCH, 2), jnp.float32),
)


def kernel(x, emb, W1, b1, W2, b2):
    pooled = _pool(x.reshape(BATCH * HIST // _CHUNK, _CHUNK), emb)
    return _mlp(pooled, W1, b1.reshape(1, HIDDEN), W2, b2.reshape(1, 2))
